# Initial kernel scaffold; baseline (speedup 1.0000x reference)
import jax
import jax.numpy as jnp
from jax.experimental import pallas as pl


def kernel(x, edge_index, edge_attr, W1a, b1a, W2a, b2a, W1m, b1m, W2m, b2m):
    raise NotImplementedError("write your pallas kernel here")


# speedup vs baseline: 2.2203x; 2.2203x over previous
import functools

import jax
import jax.numpy as jnp
from jax import lax
from jax.experimental import pallas as pl
from jax.experimental.pallas import tpu as pltpu
from jax.experimental.pallas import tpu_sc as plsc

_N = 10000
_E = 160000
_IN = 128
_NBR = 16
_OUT = 128
_H = 2
_F = 2 * _IN + _NBR
_HID = int(_F / 1.5)

_NC = 2
_NS = 16
_NW = _NC * _NS

_GW = 2 * _E // _NW
_GCH = 80
_GNCH = _GW // _GCH

_BE = 1280
_GRID = _E // _BE
_W1W = 896

_EW = _E // _NW
_SCH = 128
_SFULL = _EW // _SCH
_TAIL = _EW - _SFULL * _SCH
_SNCH = _SFULL + 1
_NACC = 10240
_NPT = _NACC // _NS

@functools.cache
def _sc_mesh():
    return plsc.VectorSubcoreMesh(
        core_axis_name="c", subcore_axis_name="s",
        num_cores=_NC, num_subcores=_NS)


@functools.cache
def _build_gather():
    @functools.partial(
        pl.kernel,
        out_type=jax.ShapeDtypeStruct((2 * _E, _IN), jnp.float32),
        mesh=_sc_mesh(),
        scratch_types=[
            pltpu.VMEM((_GW,), jnp.int32),
            pltpu.VMEM((_GCH, _IN), jnp.float32),
            pltpu.SemaphoreType.DMA,
        ],
    )
    def _gather_rows(x_hbm, idx_hbm, out_hbm, idx_v, buf, sem):
        wid = lax.axis_index("s") * _NC + lax.axis_index("c")
        base = wid * _GW
        pltpu.sync_copy(idx_hbm.at[pl.ds(base, _GW)], idx_v)

        def body(j, carry):
            off = pl.multiple_of(j * _GCH, _GCH)
            pltpu.async_copy(
                x_hbm.at[idx_v.at[pl.ds(off, _GCH)]], buf, sem).wait()
            pltpu.sync_copy(buf, out_hbm.at[pl.ds(base + off, _GCH)])
            return carry

        lax.fori_loop(0, _GNCH, body, 0)

    return _gather_rows


def _mlp_body(xi_ref, ea_ref, xj_ref, wi_ref, we_ref, wj_ref, b1_ref,
              w2a_ref, b2a_ref, w2m0_ref, w2m1_ref, b2m_ref, out_ref):
    h = jnp.dot(xi_ref[...], wi_ref[...], preferred_element_type=jnp.float32)
    h += jnp.dot(ea_ref[...], we_ref[...], preferred_element_type=jnp.float32)
    h += jnp.dot(xj_ref[...], wj_ref[...], preferred_element_type=jnp.float32)
    h += b1_ref[...]
    h = jnp.where(h >= 0, h, 0.01 * h)
    logit = jnp.dot(h, w2a_ref[...], preferred_element_type=jnp.float32)
    logit += b2a_ref[...]
    ea = jnp.exp(logit)
    w = ea / jnp.sum(ea, axis=1, keepdims=True)
    msg0 = jnp.dot(h[:, 0:256], w2m0_ref[...],
                   preferred_element_type=jnp.float32) + b2m_ref[0:1, :]
    msg1 = jnp.dot(h[:, 256:512], w2m1_ref[...],
                   preferred_element_type=jnp.float32) + b2m_ref[1:2, :]
    out_ref[...] = 0.5 * (msg0 * w[:, 0:1] + msg1 * w[:, 1:2])


def _edge_mlp(xij, edge_attr, wi, we, wj, b1, w2a, b2a, w2m0, w2m1, b2m):
    const = lambda i: (0, 0)
    return pl.pallas_call(
        _mlp_body,
        grid=(_GRID,),
        in_specs=[
            pl.BlockSpec((_BE, _IN), lambda i: (i, 0)),
            pl.BlockSpec((_BE, _NBR), lambda i: (i, 0)),
            pl.BlockSpec((_BE, _IN), lambda i: (i + _GRID, 0)),
            pl.BlockSpec((_IN, _W1W), const),
            pl.BlockSpec((_NBR, _W1W), const),
            pl.BlockSpec((_IN, _W1W), const),
            pl.BlockSpec((1, _W1W), const),
            pl.BlockSpec((_W1W, _H), const),
            pl.BlockSpec((1, _H), const),
            pl.BlockSpec((256, _OUT), const),
            pl.BlockSpec((256, _OUT), const),
            pl.BlockSpec((_H, _OUT), const),
        ],
        out_specs=pl.BlockSpec((_BE, _OUT), lambda i: (i, 0)),
        out_shape=jax.ShapeDtypeStruct((_E, _OUT), jnp.float32),
    )(xij, edge_attr, xij, wi, we, wj, b1, w2a, b2a, w2m0, w2m1, b2m)


@functools.cache
def _build_scatter():
    @functools.partial(
        pl.kernel,
        out_type=jax.ShapeDtypeStruct((_NC, _NACC, _OUT), jnp.float32),
        mesh=_sc_mesh(),
        scratch_types=[
            pltpu.VMEM((_SNCH, _SCH), jnp.int32),
            pltpu.VMEM((_SCH, _OUT), jnp.float32),
            pltpu.VMEM((_SCH, _OUT), jnp.float32),
            pltpu.VMEM_SHARED((_NACC, _OUT), jnp.float32),
            pltpu.SemaphoreType.DMA,
        ],
    )
    def _scatter_add(aggr_hbm, dstr_hbm, zeros_hbm, out_hbm,
                     idx_v, buf, tailbuf, acc, sem):
        cid = lax.axis_index("c")
        sid = lax.axis_index("s")
        wid = sid * _NC + cid
        pltpu.sync_copy(zeros_hbm, acc.at[pl.ds(sid * _NPT, _NPT)])
        pltpu.sync_copy(zeros_hbm.at[pl.ds(0, _SCH - _TAIL)],
                        tailbuf.at[pl.ds(_TAIL, _SCH - _TAIL)])
        pltpu.sync_copy(dstr_hbm.at[wid], idx_v)
        plsc.subcore_barrier()

        def body(j, carry):
            off = pl.multiple_of(j * _SCH, _SCH)
            pltpu.sync_copy(aggr_hbm.at[pl.ds(wid * _EW + off, _SCH)], buf)
            pltpu.sync_copy(buf, acc.at[idx_v.at[j]], add=True)
            return carry

        lax.fori_loop(0, _SFULL, body, 0)
        pltpu.sync_copy(aggr_hbm.at[pl.ds(wid * _EW + _SFULL * _SCH, _TAIL)],
                        tailbuf.at[pl.ds(0, _TAIL)])
        pltpu.sync_copy(tailbuf, acc.at[idx_v.at[_SFULL]], add=True)
        plsc.subcore_barrier()
        pltpu.sync_copy(acc.at[pl.ds(sid * _NPT, _NPT)],
                        out_hbm.at[cid, pl.ds(sid * _NPT, _NPT)])

    return _scatter_add


def _add_body(a_ref, b_ref, o_ref):
    o_ref[...] = a_ref[0] + b_ref[0]


def _add_partials(parts):
    bn = 2000
    return pl.pallas_call(
        _add_body,
        grid=(_N // bn,),
        in_specs=[
            pl.BlockSpec((1, bn, _OUT), lambda i: (0, i, 0)),
            pl.BlockSpec((1, bn, _OUT), lambda i: (1, i, 0)),
        ],
        out_specs=pl.BlockSpec((bn, _OUT), lambda i: (i, 0)),
        out_shape=jax.ShapeDtypeStruct((_N, _OUT), jnp.float32),
    )(parts, parts)


def kernel(x, edge_index, edge_attr, W1a, b1a, W2a, b2a, W1m, b1m, W2m, b2m):
    def chunk(w, b):
        wt = jnp.pad(w.T, ((0, 0), (0, 256 - _HID)))
        bt = jnp.pad(b, (0, 256 - _HID))
        return wt, bt

    m0w, m0b = chunk(W1m[0], b1m[0])
    m1w, m1b = chunk(W1m[1], b1m[1])
    aw = jnp.pad(jnp.concatenate([W1a[0].T, W1a[1].T], axis=1),
                 ((0, 0), (0, _W1W - 512 - 2 * _HID)))
    ab = jnp.pad(jnp.concatenate([b1a[0], b1a[1]]), (0, _W1W - 512 - 2 * _HID))
    w1 = jnp.concatenate([m0w, m1w, aw], axis=1)
    b1 = jnp.concatenate([m0b, m1b, ab])[None, :]
    wi = w1[:_IN]
    we = w1[_IN:_IN + _NBR]
    wj = w1[_IN + _NBR:]
    w2a = jnp.zeros((_W1W, _H), jnp.float32)
    w2a = w2a.at[512:512 + _HID, 0].set(W2a[0, 0])
    w2a = w2a.at[512 + _HID:512 + 2 * _HID, 1].set(W2a[1, 0])
    b2av = b2a.reshape(1, _H)
    w2m0 = jnp.pad(W2m[0].T, ((0, 256 - _HID), (0, 0)))
    w2m1 = jnp.pad(W2m[1].T, ((0, 256 - _HID), (0, 0)))

    idx_all = edge_index.reshape(-1)
    dst = edge_index[1]
    dstr = jnp.pad(dst.reshape(_NW, _EW), ((0, 0), (0, _SCH - _TAIL)),
                   constant_values=_N).reshape(_NW, _SNCH, _SCH)
    zeros = jnp.zeros((_NPT, _OUT), jnp.float32)

    xij = _gather_rows(x, idx_all)
    aggr = _edge_mlp(xij, edge_attr, wi, we, wj, b1, w2a, b2av,
                     w2m0, w2m1, b2m)
    parts = _scatter_add(aggr, dstr, zeros)
    return _add_partials(parts)


def _gather_rows(x, idx_all):
    return _build_gather()(x, idx_all)


def _scatter_add(aggr, dstr, zeros):
    return _build_scatter()(aggr, dstr, zeros)
